# Initial kernel scaffold; baseline (speedup 1.0000x reference)
import jax
import jax.numpy as jnp
from jax.experimental import pallas as pl


def kernel(x, c1_centroids, c1_weight, bn1_gamma, bn1_beta, c2_centroids, c2_weight, bn2_gamma, bn2_beta, c3_centroids, c3_weight, bn3_gamma, bn3_beta):
    raise NotImplementedError("write your pallas kernel here")


# speedup vs baseline: 3.1488x; 3.1488x over previous
import functools

import jax
import jax.numpy as jnp
from jax.experimental import pallas as pl

N = 8
HW = 56 * 56
NT = 2
TJ = HW // NT
MTOT = float(N * HW)
EPS = 1e-5


def _prep(cent, w, conv2_order):
    cb, k, sub = cent.shape
    out = w.shape[-1]
    eye = jnp.eye(cb, dtype=cent.dtype)
    if conv2_order:
        t = cent.transpose(2, 0, 1)
        a = t[:, :, :, None] * eye[None, :, None, :]
        bd = a.reshape(sub * cb, k * cb)
    else:
        t = cent.transpose(0, 2, 1)
        a = t[:, :, :, None] * eye[:, None, None, :]
        bd = a.reshape(cb * sub, k * cb)
    mmat = bd.T
    c2col = (cent * cent).sum(-1).T.reshape(k * cb, 1)
    lut = jnp.einsum('cks,cso->cko', cent, w)
    lutT = lut.transpose(2, 1, 0).reshape(out, k * cb)
    return mmat, c2col, lutT


def _vq(colsT, mmat, c2col, lutT, cb):
    cross = jnp.dot(mmat, colsT, preferred_element_type=jnp.float32)
    s = c2col - 2.0 * cross
    best = s[0:cb, :]
    code = jnp.zeros(best.shape, jnp.int32)
    for kk in range(1, 16):
        d = s[kk * cb:(kk + 1) * cb, :]
        upd = d < best
        best = jnp.where(upd, d, best)
        code = jnp.where(upd, kk, code)
    oh = jnp.concatenate(
        [(code == kk).astype(jnp.float32) for kk in range(16)], axis=0)
    return jnp.dot(lutT, oh, preferred_element_type=jnp.float32)


def _bn_ab(sref, ssref, gref, bref):
    mean = sref[...] / MTOT
    var = ssref[...] / MTOT - mean * mean
    a = gref[...] * jax.lax.rsqrt(var + EPS)
    b = bref[...] - mean * a
    return a, b


def _acc_stats(first, sref, ssref, ssum, ssq):
    @pl.when(first)
    def _():
        sref[...] = jnp.zeros_like(sref)
        ssref[...] = jnp.zeros_like(ssref)
    sref[...] += ssum
    ssref[...] += ssq


def _stage1_body(xref, mref, cref, lref, yref, sref, ssref):
    ssum = jnp.zeros((64, 1), jnp.float32)
    ssq = jnp.zeros((64, 1), jnp.float32)
    for j in range(NT):
        xb = xref[0, :, j * TJ:(j + 1) * TJ]
        y = _vq(xb, mref[...], cref[...], lref[...], 64)
        yref[0, :, j * TJ:(j + 1) * TJ] = y
        ssum += jnp.sum(y, axis=1, keepdims=True)
        ssq += jnp.sum(y * y, axis=1, keepdims=True)
    _acc_stats(pl.program_id(0) == 0, sref, ssref, ssum, ssq)


def _stage2_body(yref, s1ref, ss1ref, gref, bref, mref, cref, lref,
                 y2ref, sref, ssref):
    a, b = _bn_ab(s1ref, ss1ref, gref, bref)
    z = jnp.maximum(a * yref[0] + b, 0.0)
    zeros = jnp.zeros((64, 57), jnp.float32)
    big = jnp.concatenate([zeros, z, zeros], axis=1)
    ssum = jnp.zeros((64, 1), jnp.float32)
    ssq = jnp.zeros((64, 1), jnp.float32)
    for j in range(NT):
        w = (jax.lax.broadcasted_iota(jnp.int32, (1, TJ), 1) + j * TJ) % 56
        patches = []
        for di in range(3):
            for dj in range(3):
                off = (di - 1) * 56 + (dj - 1)
                start = 57 + off + j * TJ
                p = big[:, start:start + TJ]
                if dj == 0:
                    p = jnp.where(w >= 1, p, 0.0)
                elif dj == 2:
                    p = jnp.where(w <= 54, p, 0.0)
                patches.append(p)
        colsT = jnp.concatenate(patches, axis=0)
        y2 = _vq(colsT, mref[...], cref[...], lref[...], 64)
        y2ref[0, :, j * TJ:(j + 1) * TJ] = y2
        ssum += jnp.sum(y2, axis=1, keepdims=True)
        ssq += jnp.sum(y2 * y2, axis=1, keepdims=True)
    _acc_stats(pl.program_id(0) == 0, sref, ssref, ssum, ssq)


def _stage3_body(yref, s2ref, ss2ref, gref, bref, mref, cref, lref,
                 y3ref, sref, ssref):
    a, b = _bn_ab(s2ref, ss2ref, gref, bref)
    ssum = jnp.zeros((256, 1), jnp.float32)
    ssq = jnp.zeros((256, 1), jnp.float32)
    for j in range(NT):
        z = jnp.maximum(a * yref[0, :, j * TJ:(j + 1) * TJ] + b, 0.0)
        y3 = _vq(z, mref[...], cref[...], lref[...], 16)
        y3ref[0, :, j * TJ:(j + 1) * TJ] = y3
        ssum += jnp.sum(y3, axis=1, keepdims=True)
        ssq += jnp.sum(y3 * y3, axis=1, keepdims=True)
    _acc_stats(pl.program_id(0) == 0, sref, ssref, ssum, ssq)


def _stage4_body(y3ref, xref, s3ref, ss3ref, gref, bref, outref):
    a, b = _bn_ab(s3ref, ss3ref, gref, bref)
    outref[0] = jnp.maximum(a * y3ref[0] + b + xref[0], 0.0)


def _full(shape):
    return pl.BlockSpec(shape, lambda n: tuple(0 for _ in shape))


def _conv_call(body, nin_extra, cdim, cout):
    stat = jax.ShapeDtypeStruct((cout, 1), jnp.float32)
    img = jax.ShapeDtypeStruct((N, cout, HW), jnp.float32)
    out_specs = [
        pl.BlockSpec((1, cout, HW), lambda n: (n, 0, 0)),
        pl.BlockSpec((cout, 1), lambda n: (0, 0)),
        pl.BlockSpec((cout, 1), lambda n: (0, 0)),
    ]
    return functools.partial(
        pl.pallas_call, body, grid=(N,),
        out_shape=[img, stat, stat], out_specs=out_specs)


def kernel(x, c1_centroids, c1_weight, bn1_gamma, bn1_beta,
           c2_centroids, c2_weight, bn2_gamma, bn2_beta,
           c3_centroids, c3_weight, bn3_gamma, bn3_beta):
    xf = x.reshape(N, 256, HW)
    m1, c1c, l1 = _prep(c1_centroids, c1_weight, False)
    m2, c2c, l2 = _prep(c2_centroids, c2_weight, True)
    m3, c3c, l3 = _prep(c3_centroids, c3_weight, False)
    g1 = bn1_gamma.reshape(64, 1)
    b1 = bn1_beta.reshape(64, 1)
    g2 = bn2_gamma.reshape(64, 1)
    b2 = bn2_beta.reshape(64, 1)
    g3 = bn3_gamma.reshape(256, 1)
    b3 = bn3_beta.reshape(256, 1)

    y1, s1, ss1 = _conv_call(_stage1_body, 0, 256, 64)(
        in_specs=[pl.BlockSpec((1, 256, HW), lambda n: (n, 0, 0)),
                  _full(m1.shape), _full(c1c.shape), _full(l1.shape)],
    )(xf, m1, c1c, l1)

    y2, s2, ss2 = _conv_call(_stage2_body, 4, 576, 64)(
        in_specs=[pl.BlockSpec((1, 64, HW), lambda n: (n, 0, 0)),
                  _full(s1.shape), _full(ss1.shape),
                  _full(g1.shape), _full(b1.shape),
                  _full(m2.shape), _full(c2c.shape), _full(l2.shape)],
    )(y1, s1, ss1, g1, b1, m2, c2c, l2)

    y3, s3, ss3 = _conv_call(_stage3_body, 4, 64, 256)(
        in_specs=[pl.BlockSpec((1, 64, HW), lambda n: (n, 0, 0)),
                  _full(s2.shape), _full(ss2.shape),
                  _full(g2.shape), _full(b2.shape),
                  _full(m3.shape), _full(c3c.shape), _full(l3.shape)],
    )(y2, s2, ss2, g2, b2, m3, c3c, l3)

    out = pl.pallas_call(
        _stage4_body, grid=(N,),
        in_specs=[pl.BlockSpec((1, 256, HW), lambda n: (n, 0, 0)),
                  pl.BlockSpec((1, 256, HW), lambda n: (n, 0, 0)),
                  _full(s3.shape), _full(ss3.shape),
                  _full(g3.shape), _full(b3.shape)],
        out_specs=pl.BlockSpec((1, 256, HW), lambda n: (n, 0, 0)),
        out_shape=jax.ShapeDtypeStruct((N, 256, HW), jnp.float32),
    )(y3, xf, s3, ss3, g3, b3)

    return out.reshape(N, 256, 56, 56)


# speedup vs baseline: 3.2125x; 1.0202x over previous
import functools

import jax
import jax.numpy as jnp
from jax.experimental import pallas as pl

N = 8
HW = 56 * 56
NT = 2
TJ = HW // NT
MTOT = float(N * HW)
EPS = 1e-5


def _prep(cent, w, conv2_order):
    cb, k, sub = cent.shape
    out = w.shape[-1]
    eye = jnp.eye(cb, dtype=cent.dtype)
    if conv2_order:
        t = cent.transpose(2, 0, 1)
        a = t[:, :, :, None] * eye[None, :, None, :]
        bd = a.reshape(sub * cb, k * cb)
    else:
        t = cent.transpose(0, 2, 1)
        a = t[:, :, :, None] * eye[:, None, None, :]
        bd = a.reshape(cb * sub, k * cb)
    mmat = -2.0 * bd.T
    c2col = (cent * cent).sum(-1).T.reshape(k * cb, 1)
    lut = jnp.einsum('cks,cso->cko', cent, w)
    lutT = lut.transpose(2, 1, 0).reshape(out, k * cb)
    return mmat, c2col, lutT


def _vq(colsT, mmat, c2col, lutT, cb):
    cross = jnp.dot(mmat, colsT, preferred_element_type=jnp.float32)
    s = c2col + cross
    parts = [s[kk * cb:(kk + 1) * cb, :] for kk in range(16)]
    mins = parts
    while len(mins) > 1:
        mins = [jnp.minimum(mins[i], mins[i + 1])
                for i in range(0, len(mins), 2)]
    m = mins[0]
    oh = jnp.concatenate(
        [(p == m).astype(jnp.float32) for p in parts], axis=0)
    return jnp.dot(lutT, oh, preferred_element_type=jnp.float32)


def _bn_ab(sref, ssref, gref, bref):
    mean = sref[...] / MTOT
    var = ssref[...] / MTOT - mean * mean
    a = gref[...] * jax.lax.rsqrt(var + EPS)
    b = bref[...] - mean * a
    return a, b


def _acc_stats(first, sref, ssref, ssum, ssq):
    @pl.when(first)
    def _():
        sref[...] = jnp.zeros_like(sref)
        ssref[...] = jnp.zeros_like(ssref)
    sref[...] += ssum
    ssref[...] += ssq


def _stage1_body(xref, mref, cref, lref, yref, sref, ssref):
    ssum = jnp.zeros((64, 1), jnp.float32)
    ssq = jnp.zeros((64, 1), jnp.float32)
    for j in range(NT):
        xb = xref[0, :, j * TJ:(j + 1) * TJ]
        y = _vq(xb, mref[...], cref[...], lref[...], 64)
        yref[0, :, j * TJ:(j + 1) * TJ] = y
        ssum += jnp.sum(y, axis=1, keepdims=True)
        ssq += jnp.sum(y * y, axis=1, keepdims=True)
    _acc_stats(pl.program_id(0) == 0, sref, ssref, ssum, ssq)


def _stage2_body(yref, s1ref, ss1ref, gref, bref, mref, cref, lref,
                 y2ref, sref, ssref):
    a, b = _bn_ab(s1ref, ss1ref, gref, bref)
    z = jnp.maximum(a * yref[0] + b, 0.0)
    zeros = jnp.zeros((64, 57), jnp.float32)
    big = jnp.concatenate([zeros, z, zeros], axis=1)
    ssum = jnp.zeros((64, 1), jnp.float32)
    ssq = jnp.zeros((64, 1), jnp.float32)
    for j in range(NT):
        w = (jax.lax.broadcasted_iota(jnp.int32, (1, TJ), 1) + j * TJ) % 56
        patches = []
        for di in range(3):
            for dj in range(3):
                off = (di - 1) * 56 + (dj - 1)
                start = 57 + off + j * TJ
                p = big[:, start:start + TJ]
                if dj == 0:
                    p = jnp.where(w >= 1, p, 0.0)
                elif dj == 2:
                    p = jnp.where(w <= 54, p, 0.0)
                patches.append(p)
        colsT = jnp.concatenate(patches, axis=0)
        y2 = _vq(colsT, mref[...], cref[...], lref[...], 64)
        y2ref[0, :, j * TJ:(j + 1) * TJ] = y2
        ssum += jnp.sum(y2, axis=1, keepdims=True)
        ssq += jnp.sum(y2 * y2, axis=1, keepdims=True)
    _acc_stats(pl.program_id(0) == 0, sref, ssref, ssum, ssq)


def _stage3_body(yref, s2ref, ss2ref, gref, bref, mref, cref, lref,
                 y3ref, sref, ssref):
    a, b = _bn_ab(s2ref, ss2ref, gref, bref)
    ssum = jnp.zeros((256, 1), jnp.float32)
    ssq = jnp.zeros((256, 1), jnp.float32)
    for j in range(NT):
        z = jnp.maximum(a * yref[0, :, j * TJ:(j + 1) * TJ] + b, 0.0)
        y3 = _vq(z, mref[...], cref[...], lref[...], 16)
        y3ref[0, :, j * TJ:(j + 1) * TJ] = y3
        ssum += jnp.sum(y3, axis=1, keepdims=True)
        ssq += jnp.sum(y3 * y3, axis=1, keepdims=True)
    _acc_stats(pl.program_id(0) == 0, sref, ssref, ssum, ssq)


def _stage4_body(y3ref, xref, s3ref, ss3ref, gref, bref, outref):
    a, b = _bn_ab(s3ref, ss3ref, gref, bref)
    outref[0] = jnp.maximum(a * y3ref[0] + b + xref[0], 0.0)


def _full(shape):
    return pl.BlockSpec(shape, lambda n: tuple(0 for _ in shape))


def _conv_call(body, nin_extra, cdim, cout):
    stat = jax.ShapeDtypeStruct((cout, 1), jnp.float32)
    img = jax.ShapeDtypeStruct((N, cout, HW), jnp.float32)
    out_specs = [
        pl.BlockSpec((1, cout, HW), lambda n: (n, 0, 0)),
        pl.BlockSpec((cout, 1), lambda n: (0, 0)),
        pl.BlockSpec((cout, 1), lambda n: (0, 0)),
    ]
    return functools.partial(
        pl.pallas_call, body, grid=(N,),
        out_shape=[img, stat, stat], out_specs=out_specs)


def kernel(x, c1_centroids, c1_weight, bn1_gamma, bn1_beta,
           c2_centroids, c2_weight, bn2_gamma, bn2_beta,
           c3_centroids, c3_weight, bn3_gamma, bn3_beta):
    xf = x.reshape(N, 256, HW)
    m1, c1c, l1 = _prep(c1_centroids, c1_weight, False)
    m2, c2c, l2 = _prep(c2_centroids, c2_weight, True)
    m3, c3c, l3 = _prep(c3_centroids, c3_weight, False)
    g1 = bn1_gamma.reshape(64, 1)
    b1 = bn1_beta.reshape(64, 1)
    g2 = bn2_gamma.reshape(64, 1)
    b2 = bn2_beta.reshape(64, 1)
    g3 = bn3_gamma.reshape(256, 1)
    b3 = bn3_beta.reshape(256, 1)

    y1, s1, ss1 = _conv_call(_stage1_body, 0, 256, 64)(
        in_specs=[pl.BlockSpec((1, 256, HW), lambda n: (n, 0, 0)),
                  _full(m1.shape), _full(c1c.shape), _full(l1.shape)],
    )(xf, m1, c1c, l1)

    y2, s2, ss2 = _conv_call(_stage2_body, 4, 576, 64)(
        in_specs=[pl.BlockSpec((1, 64, HW), lambda n: (n, 0, 0)),
                  _full(s1.shape), _full(ss1.shape),
                  _full(g1.shape), _full(b1.shape),
                  _full(m2.shape), _full(c2c.shape), _full(l2.shape)],
    )(y1, s1, ss1, g1, b1, m2, c2c, l2)

    y3, s3, ss3 = _conv_call(_stage3_body, 4, 64, 256)(
        in_specs=[pl.BlockSpec((1, 64, HW), lambda n: (n, 0, 0)),
                  _full(s2.shape), _full(ss2.shape),
                  _full(g2.shape), _full(b2.shape),
                  _full(m3.shape), _full(c3c.shape), _full(l3.shape)],
    )(y2, s2, ss2, g2, b2, m3, c3c, l3)

    out = pl.pallas_call(
        _stage4_body, grid=(N,),
        in_specs=[pl.BlockSpec((1, 256, HW), lambda n: (n, 0, 0)),
                  pl.BlockSpec((1, 256, HW), lambda n: (n, 0, 0)),
                  _full(s3.shape), _full(ss3.shape),
                  _full(g3.shape), _full(b3.shape)],
        out_specs=pl.BlockSpec((1, 256, HW), lambda n: (n, 0, 0)),
        out_shape=jax.ShapeDtypeStruct((N, 256, HW), jnp.float32),
    )(y3, xf, s3, ss3, g3, b3)

    return out.reshape(N, 256, 56, 56)
